# Initial kernel scaffold; baseline (speedup 1.0000x reference)
import jax
import jax.numpy as jnp
from jax.experimental import pallas as pl


def kernel(features, env_indices, random_projection):
    raise NotImplementedError("write your pallas kernel here")


# speedup vs baseline: 4.8392x; 4.8392x over previous
import jax
import jax.numpy as jnp
from jax import lax
from jax.experimental import pallas as pl
from jax.experimental.pallas import tpu as pltpu

BATCH = 16384
D = 128
BINS = 32
ROWS = 1024
NBLK = BATCH // ROWS


def _stats_kernel(f_ref, s1_ref, s2_ref):
    b = pl.program_id(0)
    x = f_ref[...]
    s1 = jnp.sum(x, axis=0)[None, :]
    s2 = jnp.sum(x * x, axis=0)[None, :]

    @pl.when(b == 0)
    def _():
        s1_ref[...] = s1
        s2_ref[...] = s2

    @pl.when(b > 0)
    def _():
        s1_ref[...] += s1
        s2_ref[...] += s2


def _hash_kernel(f_ref, mean_ref, inv_ref, rp_ref, h_ref):
    x = (f_ref[...] - mean_ref[...]) * inv_ref[...]
    p = jnp.dot(x, rp_ref[...], preferred_element_type=jnp.float32)
    powers = jnp.left_shift(
        jnp.int32(1), lax.broadcasted_iota(jnp.int32, (1, BINS), 1)
    )
    bits = jnp.where(p > 0, powers, jnp.int32(0))
    h = jnp.sum(bits, axis=1, keepdims=True, dtype=jnp.int32)
    h_ref[...] = h


def _count_kernel(hr_ref, er_ref, hc_ref, ec_ref, out_ref):
    i = pl.program_id(0)
    j = pl.program_id(1)
    nj = pl.num_programs(1)

    @pl.when(j == 0)
    def _():
        out_ref[...] = jnp.ones((ROWS, 1), jnp.float32)

    @pl.when(j <= i)
    def _():
        hrv = hr_ref[...]
        erv = er_ref[...]
        hcv = hc_ref[...]
        ecv = ec_ref[...]
        iglob = i * ROWS + lax.broadcasted_iota(jnp.int32, (ROWS, 1), 0)
        jglob = j * ROWS + lax.broadcasted_iota(jnp.int32, (1, ROWS), 1)
        eq = (hrv == hcv) & (erv == ecv) & (jglob < iglob)
        out_ref[...] += jnp.sum(eq.astype(jnp.float32), axis=1, keepdims=True)

    @pl.when(j == nj - 1)
    def _():
        out_ref[...] = 1.0 / jnp.sqrt(out_ref[...])


def kernel(features, env_indices, random_projection):
    features = features.astype(jnp.float32)
    s1, s2 = pl.pallas_call(
        _stats_kernel,
        grid=(NBLK,),
        in_specs=[pl.BlockSpec((ROWS, D), lambda b: (b, b * 0))],
        out_specs=[
            pl.BlockSpec((1, D), lambda b: (b * 0, b * 0)),
            pl.BlockSpec((1, D), lambda b: (b * 0, b * 0)),
        ],
        out_shape=[
            jax.ShapeDtypeStruct((1, D), jnp.float32),
            jax.ShapeDtypeStruct((1, D), jnp.float32),
        ],
    )(features)

    eps_count = jnp.float32(1e-4)
    bc = jnp.float32(BATCH)
    batch_mean = s1 / bc
    batch_var = (s2 - s1 * s1 / bc) / (bc - 1.0)
    tot = eps_count + bc
    new_mean = batch_mean * (bc / tot)
    m2 = eps_count + batch_var * bc + batch_mean**2 * eps_count * bc / tot
    new_var = m2 / tot
    inv_std = 1.0 / jnp.sqrt(new_var + 1e-8)

    hashes = pl.pallas_call(
        _hash_kernel,
        grid=(NBLK,),
        in_specs=[
            pl.BlockSpec((ROWS, D), lambda b: (b, b * 0)),
            pl.BlockSpec((1, D), lambda b: (b * 0, b * 0)),
            pl.BlockSpec((1, D), lambda b: (b * 0, b * 0)),
            pl.BlockSpec((D, BINS), lambda b: (b * 0, b * 0)),
        ],
        out_specs=pl.BlockSpec((ROWS, 1), lambda b: (b, b * 0)),
        out_shape=jax.ShapeDtypeStruct((BATCH, 1), jnp.int32),
    )(features, new_mean, inv_std, random_projection.astype(jnp.float32))

    env_col = env_indices.astype(jnp.int32)[:, None]
    rewards = pl.pallas_call(
        _count_kernel,
        grid=(NBLK, NBLK),
        in_specs=[
            pl.BlockSpec((ROWS, 1), lambda i, j: (i, j * 0)),
            pl.BlockSpec((ROWS, 1), lambda i, j: (i, j * 0)),
            pl.BlockSpec((1, ROWS), lambda i, j: (i * 0, j)),
            pl.BlockSpec((1, ROWS), lambda i, j: (i * 0, j)),
        ],
        out_specs=pl.BlockSpec((ROWS, 1), lambda i, j: (i, j * 0)),
        out_shape=jax.ShapeDtypeStruct((BATCH, 1), jnp.float32),
    )(hashes, env_col, hashes.reshape(1, BATCH), env_col.reshape(1, BATCH))
    return rewards


# speedup vs baseline: 6.3674x; 1.3158x over previous
import jax
import jax.numpy as jnp
from jax import lax
from jax.experimental import pallas as pl
from jax.experimental.pallas import tpu as pltpu

BATCH = 16384
D = 128
BINS = 32
ROWS = 1024
NBLK = BATCH // ROWS


def _stats_kernel(f_ref, s1_ref, s2_ref):
    b = pl.program_id(0)
    x = f_ref[...]
    s1 = jnp.sum(x, axis=0)[None, :]
    s2 = jnp.sum(x * x, axis=0)[None, :]

    @pl.when(b == 0)
    def _():
        s1_ref[...] = s1
        s2_ref[...] = s2

    @pl.when(b > 0)
    def _():
        s1_ref[...] += s1
        s2_ref[...] += s2


def _hash_kernel(f_ref, mean_ref, inv_ref, rp_ref, h_ref):
    x = (f_ref[...] - mean_ref[...]) * inv_ref[...]
    p = jnp.dot(x, rp_ref[...], preferred_element_type=jnp.float32)
    powers = jnp.left_shift(
        jnp.int32(1), lax.broadcasted_iota(jnp.int32, (1, BINS), 1)
    )
    bits = jnp.where(p > 0, powers, jnp.int32(0))
    h = jnp.sum(bits, axis=1, keepdims=True, dtype=jnp.int32)
    h_ref[...] = h


def _count_kernel(hr_ref, er_ref, hc_ref, ec_ref, out_ref):
    i = pl.program_id(0)
    j = pl.program_id(1)
    nj = pl.num_programs(1)

    @pl.when(j == 0)
    def _():
        out_ref[...] = jnp.ones((ROWS, 1), jnp.float32)

    @pl.when(j < i)
    def _():
        eq = (hr_ref[...] == hc_ref[...]) & (er_ref[...] == ec_ref[...])
        out_ref[...] += jnp.sum(eq.astype(jnp.float32), axis=1, keepdims=True)

    @pl.when(j == i)
    def _():
        ii = lax.broadcasted_iota(jnp.int32, (ROWS, 1), 0)
        jj = lax.broadcasted_iota(jnp.int32, (1, ROWS), 1)
        eq = (hr_ref[...] == hc_ref[...]) & (er_ref[...] == ec_ref[...]) & (jj < ii)
        out_ref[...] += jnp.sum(eq.astype(jnp.float32), axis=1, keepdims=True)

    @pl.when(j == nj - 1)
    def _():
        out_ref[...] = 1.0 / jnp.sqrt(out_ref[...])


def kernel(features, env_indices, random_projection):
    features = features.astype(jnp.float32)
    s1, s2 = pl.pallas_call(
        _stats_kernel,
        grid=(NBLK,),
        in_specs=[pl.BlockSpec((ROWS, D), lambda b: (b, b * 0))],
        out_specs=[
            pl.BlockSpec((1, D), lambda b: (b * 0, b * 0)),
            pl.BlockSpec((1, D), lambda b: (b * 0, b * 0)),
        ],
        out_shape=[
            jax.ShapeDtypeStruct((1, D), jnp.float32),
            jax.ShapeDtypeStruct((1, D), jnp.float32),
        ],
    )(features)

    eps_count = jnp.float32(1e-4)
    bc = jnp.float32(BATCH)
    batch_mean = s1 / bc
    batch_var = (s2 - s1 * s1 / bc) / (bc - 1.0)
    tot = eps_count + bc
    new_mean = batch_mean * (bc / tot)
    m2 = eps_count + batch_var * bc + batch_mean**2 * eps_count * bc / tot
    new_var = m2 / tot
    inv_std = 1.0 / jnp.sqrt(new_var + 1e-8)

    hashes = pl.pallas_call(
        _hash_kernel,
        grid=(NBLK,),
        in_specs=[
            pl.BlockSpec((ROWS, D), lambda b: (b, b * 0)),
            pl.BlockSpec((1, D), lambda b: (b * 0, b * 0)),
            pl.BlockSpec((1, D), lambda b: (b * 0, b * 0)),
            pl.BlockSpec((D, BINS), lambda b: (b * 0, b * 0)),
        ],
        out_specs=pl.BlockSpec((ROWS, 1), lambda b: (b, b * 0)),
        out_shape=jax.ShapeDtypeStruct((BATCH, 1), jnp.int32),
    )(features, new_mean, inv_std, random_projection.astype(jnp.float32))

    env_col = env_indices.astype(jnp.int32)[:, None]
    rewards = pl.pallas_call(
        _count_kernel,
        grid=(NBLK, NBLK),
        in_specs=[
            pl.BlockSpec((ROWS, 1), lambda i, j: (i, j * 0)),
            pl.BlockSpec((ROWS, 1), lambda i, j: (i, j * 0)),
            pl.BlockSpec((1, ROWS), lambda i, j: (i * 0, j)),
            pl.BlockSpec((1, ROWS), lambda i, j: (i * 0, j)),
        ],
        out_specs=pl.BlockSpec((ROWS, 1), lambda i, j: (i, j * 0)),
        out_shape=jax.ShapeDtypeStruct((BATCH, 1), jnp.float32),
    )(hashes, env_col, hashes.reshape(1, BATCH), env_col.reshape(1, BATCH))
    return rewards


# speedup vs baseline: 6.3750x; 1.0012x over previous
import jax
import jax.numpy as jnp
from jax import lax
from jax.experimental import pallas as pl
from jax.experimental.pallas import tpu as pltpu

BATCH = 16384
D = 128
BINS = 32
NENV = 64
ROWS = 1024
NBLK = BATCH // ROWS


def _stats_kernel(f_ref, s1_ref, s2_ref):
    b = pl.program_id(0)
    x = f_ref[...]
    s1 = jnp.sum(x, axis=0)[None, :]
    s2 = jnp.sum(x * x, axis=0)[None, :]

    @pl.when(b == 0)
    def _():
        s1_ref[...] = s1
        s2_ref[...] = s2

    @pl.when(b > 0)
    def _():
        s1_ref[...] += s1
        s2_ref[...] += s2


def _hash_kernel(f_ref, env_ref, mean_ref, inv_ref, rp_ref, e_ref):
    x = (f_ref[...] - mean_ref[...]) * inv_ref[...]
    p = jnp.dot(x, rp_ref[...], preferred_element_type=jnp.float32)
    sign = jnp.where(p > 0, jnp.float32(1), jnp.float32(-1))
    ks = lax.broadcasted_iota(jnp.int32, (1, NENV), 1)
    onehot = jnp.where(env_ref[...] == ks, jnp.float32(8), jnp.float32(0))
    pad = jnp.zeros((ROWS, D - BINS - NENV), jnp.float32)
    e = jnp.concatenate([sign, onehot, pad], axis=1)
    e_ref[...] = e.astype(jnp.bfloat16)


def _count_kernel(ei_ref, ej_ref, out_ref):
    i = pl.program_id(0)
    j = pl.program_id(1)
    nj = pl.num_programs(1)

    @pl.when(j == 0)
    def _():
        out_ref[...] = jnp.ones((ROWS, 1), jnp.float32)

    @pl.when(j < i)
    def _():
        s = lax.dot_general(
            ei_ref[...], ej_ref[...], (((1,), (1,)), ((), ())),
            preferred_element_type=jnp.float32,
        )
        out_ref[...] += jnp.sum((s > 95.0).astype(jnp.float32), axis=1,
                                keepdims=True)

    @pl.when(j == i)
    def _():
        s = lax.dot_general(
            ei_ref[...], ej_ref[...], (((1,), (1,)), ((), ())),
            preferred_element_type=jnp.float32,
        )
        ii = lax.broadcasted_iota(jnp.int32, (ROWS, 1), 0)
        jj = lax.broadcasted_iota(jnp.int32, (1, ROWS), 1)
        eq = (s > 95.0) & (jj < ii)
        out_ref[...] += jnp.sum(eq.astype(jnp.float32), axis=1, keepdims=True)

    @pl.when(j == nj - 1)
    def _():
        out_ref[...] = 1.0 / jnp.sqrt(out_ref[...])


def kernel(features, env_indices, random_projection):
    features = features.astype(jnp.float32)
    s1, s2 = pl.pallas_call(
        _stats_kernel,
        grid=(NBLK,),
        in_specs=[pl.BlockSpec((ROWS, D), lambda b: (b, b * 0))],
        out_specs=[
            pl.BlockSpec((1, D), lambda b: (b * 0, b * 0)),
            pl.BlockSpec((1, D), lambda b: (b * 0, b * 0)),
        ],
        out_shape=[
            jax.ShapeDtypeStruct((1, D), jnp.float32),
            jax.ShapeDtypeStruct((1, D), jnp.float32),
        ],
    )(features)

    eps_count = jnp.float32(1e-4)
    bc = jnp.float32(BATCH)
    batch_mean = s1 / bc
    batch_var = (s2 - s1 * s1 / bc) / (bc - 1.0)
    tot = eps_count + bc
    new_mean = batch_mean * (bc / tot)
    m2 = eps_count + batch_var * bc + batch_mean**2 * eps_count * bc / tot
    new_var = m2 / tot
    inv_std = 1.0 / jnp.sqrt(new_var + 1e-8)

    env_col = env_indices.astype(jnp.int32)[:, None]
    enc = pl.pallas_call(
        _hash_kernel,
        grid=(NBLK,),
        in_specs=[
            pl.BlockSpec((ROWS, D), lambda b: (b, b * 0)),
            pl.BlockSpec((ROWS, 1), lambda b: (b, b * 0)),
            pl.BlockSpec((1, D), lambda b: (b * 0, b * 0)),
            pl.BlockSpec((1, D), lambda b: (b * 0, b * 0)),
            pl.BlockSpec((D, BINS), lambda b: (b * 0, b * 0)),
        ],
        out_specs=pl.BlockSpec((ROWS, D), lambda b: (b, b * 0)),
        out_shape=jax.ShapeDtypeStruct((BATCH, D), jnp.bfloat16),
    )(features, env_col, new_mean, inv_std,
      random_projection.astype(jnp.float32))

    rewards = pl.pallas_call(
        _count_kernel,
        grid=(NBLK, NBLK),
        in_specs=[
            pl.BlockSpec((ROWS, D), lambda i, j: (i, j * 0)),
            pl.BlockSpec((ROWS, D), lambda i, j: (j, i * 0)),
        ],
        out_specs=pl.BlockSpec((ROWS, 1), lambda i, j: (i, j * 0)),
        out_shape=jax.ShapeDtypeStruct((BATCH, 1), jnp.float32),
    )(enc, enc)
    return rewards
